# baseline (speedup 1.0000x reference)
import functools

import jax
import jax.numpy as jnp
from jax import lax
from jax.experimental import pallas as pl
from jax.experimental.pallas import tpu as pltpu
from jax.experimental.pallas import tpu_sc as plsc

D = 768
E = 8
DH = 2048
N = 2048
THRESHOLD = 0.2
TT = 256
NT = 2 * N // TT + E
AC = (NT + 1) * TT
MLEN = 64
NW = 32
CHUNK = N // NW
CB = 256


def _router_body(tok_ref, gw_ref, d1_ref, d2_ref, g1_ref, g2_ref,
                 meta_ref, aux_ref, oh_ref, cum_ref):
    tokens = tok_ref[...]
    logits = jnp.dot(tokens, gw_ref[...], preferred_element_type=jnp.float32)
    m = jnp.max(logits, axis=1, keepdims=True)
    ex = jnp.exp(logits - m)
    s = jnp.sum(ex, axis=1, keepdims=True)
    probs = ex / s
    z = m + jnp.log(s)

    iota = lax.broadcasted_iota(jnp.int32, (N, E), 1)
    m1 = jnp.max(probs, axis=1, keepdims=True)
    i1 = jnp.min(jnp.where(probs == m1, iota, E), axis=1, keepdims=True)
    masked = jnp.where(iota == i1, -1.0, probs)
    m2 = jnp.max(masked, axis=1, keepdims=True)
    i2 = jnp.min(jnp.where(masked == m2, iota, E), axis=1, keepdims=True)
    keep2 = m2 > THRESHOLD
    g1_ref[...] = m1
    g2_ref[...] = jnp.where(keep2, m2, 0.0)

    oh1 = (iota == i1).astype(jnp.float32)
    oh2 = jnp.where(keep2, (iota == i2).astype(jnp.float32), 0.0)
    oh = oh1 + oh2
    oh_ref[...] = oh

    tril = (lax.broadcasted_iota(jnp.int32, (CB, CB), 0)
            > lax.broadcasted_iota(jnp.int32, (CB, CB), 1)).astype(jnp.float32)

    def blk(b, base):
        rows = pl.ds(b * CB, CB)
        ohb = oh_ref[rows, :]
        cum_ref[rows, :] = base + jnp.dot(tril, ohb,
                                          preferred_element_type=jnp.float32)
        return base + jnp.sum(ohb, axis=0, keepdims=True)

    counts = lax.fori_loop(0, N // CB, blk, jnp.zeros((1, E), jnp.float32))

    pc = jnp.floor((counts + (TT - 1)) * (1.0 / TT)) * TT
    triu = (lax.broadcasted_iota(jnp.int32, (E, E), 0)
            < lax.broadcasted_iota(jnp.int32, (E, E), 1)).astype(jnp.float32)
    off_pad = jnp.dot(pc, triu, preferred_element_type=jnp.float32)
    end_pad = off_pad + pc
    nvalid = jnp.sum(pc) * (1.0 / TT)

    cum = cum_ref[...]
    pos1 = jnp.sum(oh1 * cum, axis=1, keepdims=True)
    pos2 = jnp.sum((iota == i2).astype(jnp.float32) * cum, axis=1,
                   keepdims=True)
    off1 = jnp.sum(oh1 * off_pad, axis=1, keepdims=True)
    off2 = jnp.sum((iota == i2).astype(jnp.float32) * off_pad, axis=1,
                   keepdims=True)
    d1f = off1 + pos1
    d2f = jnp.where(keep2, off2 + pos2, d1f)
    d1_ref[...] = d1f.astype(jnp.int32)
    d2_ref[...] = d2f.astype(jnp.int32)

    tif = lax.broadcasted_iota(jnp.int32, (MLEN, 1), 0).astype(jnp.float32)
    ti = tif * TT
    texp = jnp.sum((ti >= end_pad).astype(jnp.float32), axis=1,
                   keepdims=True)
    last_sel = (tif == (nvalid - 1.0)).astype(jnp.float32)
    last_texp = jnp.sum(texp * last_sel)
    valid = tif < nvalid
    texp_f = jnp.where(valid, texp, last_texp)
    xrow_f = jnp.minimum(tif, nvalid - 1.0)
    yrow_f = jnp.where(valid, tif, float(NT))
    meta_ref[pl.ds(0, MLEN), :] = texp_f.astype(jnp.int32)
    meta_ref[pl.ds(MLEN, MLEN), :] = xrow_f.astype(jnp.int32)
    meta_ref[pl.ds(2 * MLEN, MLEN), :] = yrow_f.astype(jnp.int32)
    meta_ref[pl.ds(3 * MLEN, MLEN), :] = jnp.broadcast_to(
        nvalid, (MLEN, 1)).astype(jnp.int32)

    f = counts[0] * (1.0 / N)
    P = jnp.mean(probs, axis=0)
    balance = E * jnp.sum(f * P)
    rz = jnp.mean(z * z)
    aux_ref[0, 0] = 0.01 * balance + 0.001 * rz


def _dispatch_body(tok_hbm, d1_hbm, d2_hbm, x_hbm, rows_v, d1_v, d2_v, s1, s2):
    wid = lax.axis_index("s") * 2 + lax.axis_index("c")
    base = wid * CHUNK
    pltpu.sync_copy(tok_hbm.at[pl.ds(base, CHUNK)], rows_v)
    pltpu.sync_copy(d1_hbm.at[pl.ds(base, CHUNK)], d1_v)
    pltpu.sync_copy(d2_hbm.at[pl.ds(base, CHUNK)], d2_v)
    c1 = pltpu.async_copy(rows_v, x_hbm.at[d1_v], s1)
    c2 = pltpu.async_copy(rows_v, x_hbm.at[d2_v], s2)
    c1.wait()
    c2.wait()


def _ffn_grouped_body(meta_ref, x_ref, w1_ref, b1_ref, mb_ref, w2_ref, b2_ref,
                      y_ref):
    i = pl.program_id(0)

    @pl.when(i < meta_ref[3 * MLEN])
    def _():
        x = x_ref[...].astype(jnp.bfloat16)
        h = jnp.dot(x, w1_ref[0], preferred_element_type=jnp.float32) + b1_ref[0]
        a = h[:, :DH]
        g = h[:, DH:]
        act = (a * jax.nn.gelu(g) * mb_ref[0]).astype(jnp.bfloat16)
        y_ref[...] = jnp.dot(act, w2_ref[0],
                             preferred_element_type=jnp.float32) + b2_ref[0]


def _combine_gather_body(y_hbm, d1_hbm, d2_hbm, ya_hbm, yb_hbm,
                         idx_v, rows_v, sem):
    wid = lax.axis_index("s") * 2 + lax.axis_index("c")
    base = wid * CHUNK
    pltpu.sync_copy(d1_hbm.at[pl.ds(base, CHUNK)], idx_v)
    pltpu.async_copy(y_hbm.at[idx_v], rows_v, sem).wait()
    pltpu.sync_copy(rows_v, ya_hbm.at[pl.ds(base, CHUNK)])
    pltpu.sync_copy(d2_hbm.at[pl.ds(base, CHUNK)], idx_v)
    pltpu.async_copy(y_hbm.at[idx_v], rows_v, sem).wait()
    pltpu.sync_copy(rows_v, yb_hbm.at[pl.ds(base, CHUNK)])


def _epilogue_body(ya_ref, yb_ref, g1_ref, g2_ref, lng_ref, lnb_ref, y_ref):
    y = g1_ref[...] * ya_ref[...] + g2_ref[...] * yb_ref[...]
    mu = jnp.mean(y, axis=1, keepdims=True)
    yc = y - mu
    var = jnp.mean(yc * yc, axis=1, keepdims=True)
    y_ref[...] = yc * lax.rsqrt(var + 1e-5) * lng_ref[...] + lnb_ref[...]


def kernel(x, gate_W, W1, b1, mult_bias, W2, b2, ln_g, ln_b):
    tokens = x.reshape(N, D)

    d1, d2, g1, g2, meta, aux = pl.pallas_call(
        _router_body,
        out_shape=[
            jax.ShapeDtypeStruct((N, 1), jnp.int32),
            jax.ShapeDtypeStruct((N, 1), jnp.int32),
            jax.ShapeDtypeStruct((N, 1), jnp.float32),
            jax.ShapeDtypeStruct((N, 1), jnp.float32),
            jax.ShapeDtypeStruct((4 * MLEN, 1), jnp.int32),
            jax.ShapeDtypeStruct((1, 1), jnp.float32),
        ],
        out_specs=[
            pl.BlockSpec(memory_space=pltpu.VMEM),
            pl.BlockSpec(memory_space=pltpu.VMEM),
            pl.BlockSpec(memory_space=pltpu.VMEM),
            pl.BlockSpec(memory_space=pltpu.VMEM),
            pl.BlockSpec(memory_space=pltpu.VMEM),
            pl.BlockSpec(memory_space=pltpu.SMEM),
        ],
        scratch_shapes=[
            pltpu.VMEM((N, E), jnp.float32),
            pltpu.VMEM((N, E), jnp.float32),
        ],
    )(tokens, gate_W)

    d1_flat = d1.reshape(N)
    d2_flat = d2.reshape(N)

    dispatch = functools.partial(
        pl.kernel,
        mesh=plsc.VectorSubcoreMesh(core_axis_name="c", subcore_axis_name="s"),
        out_type=jax.ShapeDtypeStruct((AC, D), jnp.float32),
        scratch_types=[
            pltpu.VMEM((CHUNK, D), jnp.float32),
            pltpu.VMEM((CHUNK,), jnp.int32),
            pltpu.VMEM((CHUNK,), jnp.int32),
            pltpu.SemaphoreType.DMA,
            pltpu.SemaphoreType.DMA,
        ],
    )(_dispatch_body)
    x_cap = dispatch(tokens, d1_flat, d2_flat)

    grid_spec = pltpu.PrefetchScalarGridSpec(
        num_scalar_prefetch=1,
        grid=(NT,),
        in_specs=[
            pl.BlockSpec((TT, D), lambda i, mt: (mt[MLEN + i], 0)),
            pl.BlockSpec((1, D, 2 * DH), lambda i, mt: (mt[i], 0, 0)),
            pl.BlockSpec((1, 1, 2 * DH), lambda i, mt: (mt[i], 0, 0)),
            pl.BlockSpec((1, 1, DH), lambda i, mt: (mt[i], 0, 0)),
            pl.BlockSpec((1, DH, D), lambda i, mt: (mt[i], 0, 0)),
            pl.BlockSpec((1, 1, D), lambda i, mt: (mt[i], 0, 0)),
        ],
        out_specs=pl.BlockSpec((TT, D), lambda i, mt: (mt[2 * MLEN + i], 0)),
    )
    y_cap = pl.pallas_call(
        _ffn_grouped_body,
        grid_spec=grid_spec,
        out_shape=jax.ShapeDtypeStruct((AC, D), jnp.float32),
        compiler_params=pltpu.CompilerParams(
            dimension_semantics=("arbitrary",)),
    )(meta.reshape(4 * MLEN), x_cap, W1.astype(jnp.bfloat16),
      b1.reshape(E, 1, 2 * DH), mult_bias.reshape(E, 1, DH),
      W2.astype(jnp.bfloat16), b2.reshape(E, 1, D))

    combine = functools.partial(
        pl.kernel,
        mesh=plsc.VectorSubcoreMesh(core_axis_name="c", subcore_axis_name="s"),
        out_type=[
            jax.ShapeDtypeStruct((N, D), jnp.float32),
            jax.ShapeDtypeStruct((N, D), jnp.float32),
        ],
        scratch_types=[
            pltpu.VMEM((CHUNK,), jnp.int32),
            pltpu.VMEM((CHUNK, D), jnp.float32),
            pltpu.SemaphoreType.DMA,
        ],
    )(_combine_gather_body)
    ya, yb = combine(y_cap, d1_flat, d2_flat)

    y = pl.pallas_call(
        _epilogue_body,
        grid=(N // TT,),
        in_specs=[
            pl.BlockSpec((TT, D), lambda t: (t, 0)),
            pl.BlockSpec((TT, D), lambda t: (t, 0)),
            pl.BlockSpec((TT, 1), lambda t: (t, 0)),
            pl.BlockSpec((TT, 1), lambda t: (t, 0)),
            pl.BlockSpec((1, D), lambda t: (0, 0)),
            pl.BlockSpec((1, D), lambda t: (0, 0)),
        ],
        out_specs=pl.BlockSpec((TT, D), lambda t: (t, 0)),
        out_shape=jax.ShapeDtypeStruct((N, D), jnp.float32),
        compiler_params=pltpu.CompilerParams(
            dimension_semantics=("arbitrary",)),
    )(ya, yb, g1, g2, ln_g.reshape(1, D), ln_b.reshape(1, D))

    return y.reshape(x.shape), aux.reshape(())


# speedup vs baseline: 1.3165x; 1.3165x over previous
import functools

import jax
import jax.numpy as jnp
from jax import lax
from jax.experimental import pallas as pl
from jax.experimental.pallas import tpu as pltpu
from jax.experimental.pallas import tpu_sc as plsc

D = 768
E = 8
DH = 2048
N = 2048
THRESHOLD = 0.2
TT = 256
NT = 2 * N // TT + E
AC = (NT + 1) * TT
MLEN = 64
NW = 32
CHUNK = N // NW
CB = 256


def _router_body(tok_ref, gw_ref, d1_ref, d2_ref, g1_ref, g2_ref,
                 meta_ref, aux_ref):
    tokens = tok_ref[...]
    logits = jnp.dot(tokens, gw_ref[...], preferred_element_type=jnp.float32)
    m = jnp.max(logits, axis=1, keepdims=True)
    ex = jnp.exp(logits - m)
    s = jnp.sum(ex, axis=1, keepdims=True)
    probs = ex / s
    z = m + jnp.log(s)

    iota = lax.broadcasted_iota(jnp.int32, (N, E), 1)
    m1 = jnp.max(probs, axis=1, keepdims=True)
    i1 = jnp.min(jnp.where(probs == m1, iota, E), axis=1, keepdims=True)
    masked = jnp.where(iota == i1, -1.0, probs)
    m2 = jnp.max(masked, axis=1, keepdims=True)
    i2 = jnp.min(jnp.where(masked == m2, iota, E), axis=1, keepdims=True)
    keep2 = m2 > THRESHOLD
    g1b = jnp.broadcast_to(m1, (N, 16))
    g2b = jnp.broadcast_to(jnp.where(keep2, m2, 0.0), (N, 16))
    g1_ref[...] = g1b
    g2_ref[...] = g2b

    oh1 = (iota == i1).astype(jnp.float32)
    oh2 = jnp.where(keep2, (iota == i2).astype(jnp.float32), 0.0)
    oh = oh1 + oh2

    c = oh
    k = 1
    while k < N:
        c = c + jnp.concatenate(
            [jnp.zeros((k, E), jnp.float32), c[:N - k, :]], axis=0)
        k *= 2
    cum = c - oh
    counts = c[N - 1:N, :]

    pc = jnp.floor((counts + (TT - 1)) * (1.0 / TT)) * TT
    triu = (lax.broadcasted_iota(jnp.int32, (E, E), 0)
            < lax.broadcasted_iota(jnp.int32, (E, E), 1)).astype(jnp.float32)
    off_pad = jnp.dot(pc, triu, preferred_element_type=jnp.float32)
    end_pad = off_pad + pc
    nvalid = jnp.sum(pc) * (1.0 / TT)

    pos1 = jnp.sum(oh1 * cum, axis=1, keepdims=True)
    pos2 = jnp.sum((iota == i2).astype(jnp.float32) * cum, axis=1,
                   keepdims=True)
    off1 = jnp.sum(oh1 * off_pad, axis=1, keepdims=True)
    off2 = jnp.sum((iota == i2).astype(jnp.float32) * off_pad, axis=1,
                   keepdims=True)
    d1f = off1 + pos1
    d2f = jnp.where(keep2, off2 + pos2, d1f)
    d1_ref[...] = d1f.astype(jnp.int32)
    d2_ref[...] = d2f.astype(jnp.int32)

    tif = lax.broadcasted_iota(jnp.int32, (MLEN, 1), 0).astype(jnp.float32)
    ti = tif * TT
    texp = jnp.sum((ti >= end_pad).astype(jnp.float32), axis=1,
                   keepdims=True)
    last_sel = (tif == (nvalid - 1.0)).astype(jnp.float32)
    last_texp = jnp.sum(texp * last_sel)
    valid = tif < nvalid
    texp_f = jnp.where(valid, texp, last_texp)
    xrow_f = jnp.minimum(tif, nvalid - 1.0)
    yrow_f = jnp.where(valid, tif, float(NT))
    meta_ref[pl.ds(0, MLEN), :] = texp_f.astype(jnp.int32)
    meta_ref[pl.ds(MLEN, MLEN), :] = xrow_f.astype(jnp.int32)
    meta_ref[pl.ds(2 * MLEN, MLEN), :] = yrow_f.astype(jnp.int32)
    meta_ref[pl.ds(3 * MLEN, MLEN), :] = jnp.broadcast_to(
        nvalid, (MLEN, 1)).astype(jnp.int32)

    f = counts[0] * (1.0 / N)
    P = jnp.mean(probs, axis=0)
    balance = E * jnp.sum(f * P)
    rz = jnp.mean(z * z)
    aux_ref[0, 0] = 0.01 * balance + 0.001 * rz


def _dispatch_body(tok_hbm, d1_hbm, d2_hbm, x_hbm, rows_v, d1_v, d2_v, s1, s2):
    wid = lax.axis_index("s") * 2 + lax.axis_index("c")
    base = wid * CHUNK
    pltpu.sync_copy(tok_hbm.at[pl.ds(base, CHUNK)], rows_v)
    pltpu.sync_copy(d1_hbm.at[pl.ds(base, CHUNK)], d1_v)
    pltpu.sync_copy(d2_hbm.at[pl.ds(base, CHUNK)], d2_v)
    c1 = pltpu.async_copy(rows_v, x_hbm.at[d1_v], s1)
    c2 = pltpu.async_copy(rows_v, x_hbm.at[d2_v], s2)
    c1.wait()
    c2.wait()


def _ffn_grouped_body(meta_ref, x_ref, w1_ref, b1_ref, mb_ref, w2_ref, b2_ref,
                      y_ref):
    i = pl.program_id(0)

    @pl.when(i < meta_ref[3 * MLEN])
    def _():
        x = x_ref[...]
        h = jnp.dot(x, w1_ref[0], preferred_element_type=jnp.float32) + b1_ref[0]
        a = h[:, :DH]
        g = h[:, DH:]
        act = a * jax.nn.gelu(g) * mb_ref[0]
        y_ref[...] = jnp.dot(act, w2_ref[0],
                             preferred_element_type=jnp.float32) + b2_ref[0]


def _combine_gather_body(y_hbm, d1_hbm, d2_hbm, g1_hbm, g2_hbm, mo_hbm,
                         idx_v, rows_a, rows_b, g1_v, g2_v, sem, semb):
    wid = lax.axis_index("s") * 2 + lax.axis_index("c")
    base = wid * CHUNK
    pltpu.sync_copy(d1_hbm.at[pl.ds(base, CHUNK)], idx_v)
    ca = pltpu.async_copy(y_hbm.at[idx_v], rows_a, sem)
    pltpu.sync_copy(g1_hbm.at[pl.ds(base, CHUNK)], g1_v)
    pltpu.sync_copy(g2_hbm.at[pl.ds(base, CHUNK)], g2_v)
    ca.wait()
    pltpu.sync_copy(d2_hbm.at[pl.ds(base, CHUNK)], idx_v)
    pltpu.async_copy(y_hbm.at[idx_v], rows_b, semb).wait()

    def tok(i, carry):
        ga = g1_v[i, :]
        gb = g2_v[i, :]
        for j in range(D // 16):
            sl = pl.ds(16 * j, 16)
            rows_a[i, sl] = ga * rows_a[i, sl] + gb * rows_b[i, sl]
        return carry

    lax.fori_loop(0, CHUNK, tok, 0)
    pltpu.sync_copy(rows_a, mo_hbm.at[pl.ds(base, CHUNK)])


def _epilogue_body(mo_ref, lng_ref, lnb_ref, y_ref):
    y = mo_ref[...]
    mu = jnp.mean(y, axis=1, keepdims=True)
    yc = y - mu
    var = jnp.mean(yc * yc, axis=1, keepdims=True)
    y_ref[...] = yc * lax.rsqrt(var + 1e-5) * lng_ref[...] + lnb_ref[...]


def kernel(x, gate_W, W1, b1, mult_bias, W2, b2, ln_g, ln_b):
    tokens = x.reshape(N, D)

    d1, d2, g1, g2, meta, aux = pl.pallas_call(
        _router_body,
        out_shape=[
            jax.ShapeDtypeStruct((N, 1), jnp.int32),
            jax.ShapeDtypeStruct((N, 1), jnp.int32),
            jax.ShapeDtypeStruct((N, 16), jnp.float32),
            jax.ShapeDtypeStruct((N, 16), jnp.float32),
            jax.ShapeDtypeStruct((4 * MLEN, 1), jnp.int32),
            jax.ShapeDtypeStruct((1, 1), jnp.float32),
        ],
        out_specs=[
            pl.BlockSpec(memory_space=pltpu.VMEM),
            pl.BlockSpec(memory_space=pltpu.VMEM),
            pl.BlockSpec(memory_space=pltpu.VMEM),
            pl.BlockSpec(memory_space=pltpu.VMEM),
            pl.BlockSpec(memory_space=pltpu.VMEM),
            pl.BlockSpec(memory_space=pltpu.SMEM),
        ],
    )(tokens, gate_W)

    d1_flat = d1.reshape(N)
    d2_flat = d2.reshape(N)

    dispatch = functools.partial(
        pl.kernel,
        mesh=plsc.VectorSubcoreMesh(core_axis_name="c", subcore_axis_name="s"),
        out_type=jax.ShapeDtypeStruct((AC, D), jnp.float32),
        scratch_types=[
            pltpu.VMEM((CHUNK, D), jnp.float32),
            pltpu.VMEM((CHUNK,), jnp.int32),
            pltpu.VMEM((CHUNK,), jnp.int32),
            pltpu.SemaphoreType.DMA,
            pltpu.SemaphoreType.DMA,
        ],
    )(_dispatch_body)
    x_cap = dispatch(tokens, d1_flat, d2_flat)

    grid_spec = pltpu.PrefetchScalarGridSpec(
        num_scalar_prefetch=1,
        grid=(NT,),
        in_specs=[
            pl.BlockSpec((TT, D), lambda i, mt: (mt[MLEN + i], 0)),
            pl.BlockSpec((1, D, 2 * DH), lambda i, mt: (mt[i], 0, 0)),
            pl.BlockSpec((1, 1, 2 * DH), lambda i, mt: (mt[i], 0, 0)),
            pl.BlockSpec((1, 1, DH), lambda i, mt: (mt[i], 0, 0)),
            pl.BlockSpec((1, DH, D), lambda i, mt: (mt[i], 0, 0)),
            pl.BlockSpec((1, 1, D), lambda i, mt: (mt[i], 0, 0)),
        ],
        out_specs=pl.BlockSpec((TT, D), lambda i, mt: (mt[2 * MLEN + i], 0)),
    )
    y_cap = pl.pallas_call(
        _ffn_grouped_body,
        grid_spec=grid_spec,
        out_shape=jax.ShapeDtypeStruct((AC, D), jnp.float32),
        compiler_params=pltpu.CompilerParams(
            dimension_semantics=("arbitrary",)),
    )(meta.reshape(4 * MLEN), x_cap, W1, b1.reshape(E, 1, 2 * DH),
      mult_bias.reshape(E, 1, DH), W2, b2.reshape(E, 1, D))

    combine = functools.partial(
        pl.kernel,
        mesh=plsc.VectorSubcoreMesh(core_axis_name="c", subcore_axis_name="s"),
        out_type=jax.ShapeDtypeStruct((N, D), jnp.float32),
        scratch_types=[
            pltpu.VMEM((CHUNK,), jnp.int32),
            pltpu.VMEM((CHUNK, D), jnp.float32),
            pltpu.VMEM((CHUNK, D), jnp.float32),
            pltpu.VMEM((CHUNK, 16), jnp.float32),
            pltpu.VMEM((CHUNK, 16), jnp.float32),
            pltpu.SemaphoreType.DMA,
            pltpu.SemaphoreType.DMA,
        ],
    )(_combine_gather_body)
    mo = combine(y_cap, d1_flat, d2_flat, g1, g2)

    y = pl.pallas_call(
        _epilogue_body,
        grid=(N // TT,),
        in_specs=[
            pl.BlockSpec((TT, D), lambda t: (t, 0)),
            pl.BlockSpec((1, D), lambda t: (0, 0)),
            pl.BlockSpec((1, D), lambda t: (0, 0)),
        ],
        out_specs=pl.BlockSpec((TT, D), lambda t: (t, 0)),
        out_shape=jax.ShapeDtypeStruct((N, D), jnp.float32),
        compiler_params=pltpu.CompilerParams(
            dimension_semantics=("arbitrary",)),
    )(mo, ln_g.reshape(1, D), ln_b.reshape(1, D))

    return y.reshape(x.shape), aux.reshape(())
